# baseline (speedup 1.0000x reference)
import functools

import jax
import jax.numpy as jnp
from jax import lax
from jax.experimental import pallas as pl
from jax.experimental.pallas import tpu as pltpu
from jax.experimental.pallas import tpu_sc as plsc

K = 8192
D = 256
N = 8192
NH = N // 2
T = 256
GRIDH = NH // T
COMMITMENT_COST = 0.25

NC = 2
NS = 16
NW = NC * NS
BPW = NH // NW
CHUNK = 64
LANES = 16


def _tc_body(half, *refs):
    if half == 0:
        x_ref, emb_ref, hin_ref, enc_ref, idx_ref, hout_ref, perp_ref, hist_acc = refs
    else:
        x_ref, emb_ref, hin_ref, _, enc_ref, idx_ref, hout_ref, perp_ref, hist_acc = refs
    i = pl.program_id(0)
    x = x_ref[...]
    e = emb_ref[...]

    @pl.when(i == 0)
    def _init():
        hist_acc[...] = hin_ref[...]

    x_norm = jnp.sum(x * x, axis=1, keepdims=True)
    e_norm = jnp.sum(e * e, axis=1)
    x2b = (x + x).astype(jnp.bfloat16)
    eb = e.astype(jnp.bfloat16)
    prod2 = jax.lax.dot_general(x2b, eb, (((1,), (1,)), ((), ())),
                                preferred_element_type=jnp.float32)
    dist = (x_norm + e_norm[None, :]) - prod2

    mn = jnp.min(dist, axis=1, keepdims=True)
    iota = jax.lax.broadcasted_iota(jnp.int32, (T, K), 1)
    idx = jnp.min(jnp.where(dist == mn, iota, K), axis=1)
    idx_ref[...] = idx[:, None]
    enc = (iota == idx[:, None]).astype(jnp.float32)
    enc_ref[...] = enc
    hist_acc[...] += jnp.sum(enc, axis=0)[None, :]

    @pl.when(i == GRIDH - 1)
    def _fini():
        hout_ref[...] = hist_acc[...]
        if half == 1:
            avg = hist_acc[...] * jnp.float32(1.0 / N)
            ent = jnp.sum(avg * jnp.log(avg + 1e-10))
            perp_ref[0, 0] = jnp.exp(-ent)
        else:
            perp_ref[0, 0] = jnp.float32(0.0)


def _tc_half(half, z_half, embedding, hist_in, enc_prev=None):
    in_specs = [
        pl.BlockSpec((T, D), lambda i: (i, 0)),
        pl.BlockSpec((K, D), lambda i: (0, 0)),
        pl.BlockSpec((1, K), lambda i: (0, 0)),
    ]
    args = [z_half, embedding, hist_in]
    aliases = {}
    if half == 1:
        in_specs.append(pl.BlockSpec(memory_space=pl.ANY))
        args.append(enc_prev)
        aliases = {3: 0}
    return pl.pallas_call(
        functools.partial(_tc_body, half),
        grid=(GRIDH,),
        in_specs=in_specs,
        out_specs=[
            pl.BlockSpec((T, K), lambda i, half=half: (half * GRIDH + i, 0)),
            pl.BlockSpec((T, 1), lambda i: (i, 0)),
            pl.BlockSpec((1, K), lambda i: (0, 0)),
            pl.BlockSpec((1, 1), lambda i: (0, 0), memory_space=pltpu.SMEM),
        ],
        out_shape=[
            jax.ShapeDtypeStruct((N, K), jnp.float32),
            jax.ShapeDtypeStruct((NH, 1), jnp.int32),
            jax.ShapeDtypeStruct((1, K), jnp.float32),
            jax.ShapeDtypeStruct((1, 1), jnp.float32),
        ],
        input_output_aliases=aliases,
        scratch_shapes=[
            pltpu.VMEM((1, K), jnp.float32),
        ],
    )(*args)


def _sc_kernel_fn(ebf_hbm, idx_hbm, x_hbm, st_hbm, part_hbm,
                  idx_v, zq_v, x_v, vec_v, sem):
    wid = lax.axis_index("s") * NC + lax.axis_index("c")
    base = wid * BPW
    pltpu.sync_copy(idx_hbm.at[pl.ds(base, BPW)], idx_v)

    acc = jnp.zeros((LANES,), jnp.float32)
    for c in range(BPW // CHUNK):
        pltpu.async_copy(ebf_hbm.at[idx_v.at[pl.ds(c * CHUNK, CHUNK)]],
                         zq_v, sem).wait()
        pltpu.sync_copy(x_hbm.at[pl.ds(base + c * CHUNK, CHUNK)], x_v)

        def row_body(r, a):
            for j in range(D // LANES):
                sl = pl.ds(j * LANES, LANES)
                zq = zq_v[r, sl]
                xv = x_v[r, sl]
                t = zq - xv
                zq_v[r, sl] = xv + t
                a = a + t * t
            return a

        acc = lax.fori_loop(0, CHUNK, row_body, acc)
        pltpu.sync_copy(zq_v, st_hbm.at[pl.ds(base + c * CHUNK, CHUNK)])

    vec_v[...] = acc
    pltpu.sync_copy(vec_v, part_hbm.at[wid])


_sc_quantize = functools.partial(
    pl.kernel,
    mesh=plsc.VectorSubcoreMesh(core_axis_name="c", subcore_axis_name="s"),
    out_type=[
        jax.ShapeDtypeStruct((NH, D), jnp.float32),
        jax.ShapeDtypeStruct((NW, LANES), jnp.float32),
    ],
    scratch_types=[
        pltpu.VMEM((BPW,), jnp.int32),
        pltpu.VMEM((CHUNK, D), jnp.float32),
        pltpu.VMEM((CHUNK, D), jnp.float32),
        pltpu.VMEM((LANES,), jnp.float32),
        pltpu.SemaphoreType.DMA,
    ],
)(_sc_kernel_fn)


def kernel(z_e, embedding):
    B, Dm, H, W = z_e.shape
    z = jnp.transpose(z_e, (0, 2, 3, 1)).reshape(N, D)
    hist0 = jnp.zeros((1, K), jnp.float32)
    enc_a, idx_a, hist_a, _ = _tc_half(0, z[:NH], embedding, hist0)
    enc, idx_b, _, perp = _tc_half(1, z[NH:], embedding, hist_a, enc_a)
    ebf = embedding.astype(jnp.bfloat16).astype(jnp.float32)
    st_a, parts_a = _sc_quantize(ebf, idx_a.reshape(NH), z[:NH])
    st_b, parts_b = _sc_quantize(ebf, idx_b.reshape(NH), z[NH:])
    m = (jnp.sum(parts_a) + jnp.sum(parts_b)) / jnp.float32(N * D)
    loss = m + COMMITMENT_COST * m
    st = jnp.concatenate([st_a, st_b], axis=0)
    out = jnp.transpose(st.reshape(B, H, W, Dm), (0, 3, 1, 2))
    return out, loss, perp[0, 0], enc


# speedup vs baseline: 1.1214x; 1.1214x over previous
import functools

import jax
import jax.numpy as jnp
from jax import lax
from jax.experimental import pallas as pl
from jax.experimental.pallas import tpu as pltpu
from jax.experimental.pallas import tpu_sc as plsc

K = 8192
D = 256
N = 8192
T = 256
GRID = N // T
COMMITMENT_COST = 0.25

NC = 2
NS = 16
NW = NC * NS
BPW = N // NW
CHUNK = 64
NCHUNK = BPW // CHUNK
LANES = 16


def _tc_body(x_ref, emb_ref, enc_ref, idx_ref, perp_ref, hist_acc):
    i = pl.program_id(0)
    x = x_ref[...]
    e = emb_ref[...]

    @pl.when(i == 0)
    def _init():
        hist_acc[...] = jnp.zeros_like(hist_acc)

    x_norm = jnp.sum(x * x, axis=1, keepdims=True)
    e_norm = jnp.sum(e * e, axis=1)
    x2b = (x + x).astype(jnp.bfloat16)
    eb = e.astype(jnp.bfloat16)
    prod2 = jax.lax.dot_general(x2b, eb, (((1,), (1,)), ((), ())),
                                preferred_element_type=jnp.float32)
    dist = (x_norm + e_norm[None, :]) - prod2

    mn = jnp.min(dist, axis=1, keepdims=True)
    iota = jax.lax.broadcasted_iota(jnp.int32, (T, K), 1)
    idx = jnp.min(jnp.where(dist == mn, iota, K), axis=1)
    idx_ref[...] = idx[:, None]
    enc = (iota == idx[:, None]).astype(jnp.float32)
    enc_ref[...] = enc
    hist_acc[...] += jnp.sum(enc, axis=0)[None, :]

    @pl.when(i == GRID - 1)
    def _fini():
        avg = hist_acc[...] * jnp.float32(1.0 / N)
        ent = jnp.sum(avg * jnp.log(avg + 1e-10))
        perp_ref[0, 0] = jnp.exp(-ent)


def _sc_kernel_fn(ebf_hbm, idx_hbm, x_hbm, st_hbm, part_hbm,
                  idx_v, zq0_v, zq1_v, x_v, vec_v, sem0, sem1):
    wid = lax.axis_index("s") * NC + lax.axis_index("c")
    base = wid * BPW
    pltpu.sync_copy(idx_hbm.at[pl.ds(base, BPW)], idx_v)

    zq_bufs = (zq0_v, zq1_v)
    sems = (sem0, sem1)
    pltpu.async_copy(ebf_hbm.at[idx_v.at[pl.ds(0, CHUNK)]], zq0_v, sem0)

    acc = jnp.zeros((LANES,), jnp.float32)
    for c in range(NCHUNK):
        zq_v = zq_bufs[c % 2]
        if c + 1 < NCHUNK:
            pltpu.async_copy(
                ebf_hbm.at[idx_v.at[pl.ds((c + 1) * CHUNK, CHUNK)]],
                zq_bufs[(c + 1) % 2], sems[(c + 1) % 2])
        pltpu.sync_copy(x_hbm.at[pl.ds(base + c * CHUNK, CHUNK)], x_v)
        pltpu.make_async_copy(
            ebf_hbm.at[idx_v.at[pl.ds(c * CHUNK, CHUNK)]],
            zq_v, sems[c % 2]).wait()

        def row_body(r, a):
            for j in range(D // LANES):
                sl = pl.ds(j * LANES, LANES)
                zq = zq_v[r, sl]
                xv = x_v[r, sl]
                t = zq - xv
                zq_v[r, sl] = xv + t
                a = a + t * t
            return a

        acc = lax.fori_loop(0, CHUNK, row_body, acc)
        pltpu.sync_copy(zq_v, st_hbm.at[pl.ds(base + c * CHUNK, CHUNK)])

    vec_v[...] = acc
    pltpu.sync_copy(vec_v, part_hbm.at[wid])


_sc_quantize = functools.partial(
    pl.kernel,
    mesh=plsc.VectorSubcoreMesh(core_axis_name="c", subcore_axis_name="s"),
    out_type=[
        jax.ShapeDtypeStruct((N, D), jnp.float32),
        jax.ShapeDtypeStruct((NW, LANES), jnp.float32),
    ],
    scratch_types=[
        pltpu.VMEM((BPW,), jnp.int32),
        pltpu.VMEM((CHUNK, D), jnp.float32),
        pltpu.VMEM((CHUNK, D), jnp.float32),
        pltpu.VMEM((CHUNK, D), jnp.float32),
        pltpu.VMEM((LANES,), jnp.float32),
        pltpu.SemaphoreType.DMA,
        pltpu.SemaphoreType.DMA,
    ],
)(_sc_kernel_fn)


def kernel(z_e, embedding):
    B, Dm, H, W = z_e.shape
    z = jnp.transpose(z_e, (0, 2, 3, 1)).reshape(N, D)
    enc, idxo, perp = pl.pallas_call(
        _tc_body,
        grid=(GRID,),
        in_specs=[
            pl.BlockSpec((T, D), lambda i: (i, 0)),
            pl.BlockSpec((K, D), lambda i: (0, 0)),
        ],
        out_specs=[
            pl.BlockSpec((T, K), lambda i: (i, 0)),
            pl.BlockSpec((T, 1), lambda i: (i, 0)),
            pl.BlockSpec((1, 1), lambda i: (0, 0), memory_space=pltpu.SMEM),
        ],
        out_shape=[
            jax.ShapeDtypeStruct((N, K), jnp.float32),
            jax.ShapeDtypeStruct((N, 1), jnp.int32),
            jax.ShapeDtypeStruct((1, 1), jnp.float32),
        ],
        scratch_shapes=[
            pltpu.VMEM((1, K), jnp.float32),
        ],
    )(z, embedding)
    ebf = embedding.astype(jnp.bfloat16).astype(jnp.float32)
    st, loss_parts = _sc_quantize(ebf, idxo.reshape(N), z)
    m = jnp.sum(loss_parts) / jnp.float32(N * D)
    loss = m + COMMITMENT_COST * m
    out = jnp.transpose(st.reshape(B, H, W, Dm), (0, 3, 1, 2))
    return out, loss, perp[0, 0], enc


# speedup vs baseline: 1.1448x; 1.0208x over previous
import functools

import jax
import jax.numpy as jnp
from jax import lax
from jax.experimental import pallas as pl
from jax.experimental.pallas import tpu as pltpu
from jax.experimental.pallas import tpu_sc as plsc

K = 8192
D = 256
N = 8192
T = 256
GRID = N // T
COMMITMENT_COST = 0.25

NC = 2
NS = 16
NW = NC * NS
BPW = N // NW
CHUNK = 64
NCHUNK = BPW // CHUNK
LANES = 16


def _tc_body(x_ref, emb_ref, enc_ref, idx_ref, perp_ref, hist_acc):
    i = pl.program_id(0)
    x = x_ref[...]
    e = emb_ref[...]

    @pl.when(i == 0)
    def _init():
        hist_acc[...] = jnp.zeros_like(hist_acc)

    x_norm = jnp.sum(x * x, axis=1, keepdims=True)
    e_norm = jnp.sum(e * e, axis=1)
    x2b = (x + x).astype(jnp.bfloat16)
    eb = e.astype(jnp.bfloat16)
    prod2 = jax.lax.dot_general(x2b, eb, (((1,), (1,)), ((), ())),
                                preferred_element_type=jnp.float32)
    dist = (x_norm + e_norm[None, :]) - prod2

    mn = jnp.min(dist, axis=1, keepdims=True)
    iota = jax.lax.broadcasted_iota(jnp.int32, (T, K), 1)
    idx = jnp.min(jnp.where(dist == mn, iota, K), axis=1)
    idx_ref[...] = idx[:, None]
    enc = (iota == idx[:, None]).astype(jnp.float32)
    enc_ref[...] = enc
    hist_acc[...] += jnp.sum(enc, axis=0)[None, :]

    @pl.when(i == GRID - 1)
    def _fini():
        avg = hist_acc[...] * jnp.float32(1.0 / N)
        ent = jnp.sum(avg * jnp.log(avg + 1e-10))
        perp_ref[0, 0] = jnp.exp(-ent)


def _sc_kernel_fn(ebf_hbm, idx_hbm, x_hbm, st_hbm, part_hbm,
                  idx_v, zq0_v, zq1_v, x_v, vec_v, sem0, sem1):
    wid = lax.axis_index("s") * NC + lax.axis_index("c")
    base = wid * BPW
    pltpu.sync_copy(idx_hbm.at[pl.ds(base, BPW)], idx_v)

    zq_bufs = (zq0_v, zq1_v)
    sems = (sem0, sem1)
    pltpu.async_copy(ebf_hbm.at[idx_v.at[pl.ds(0, CHUNK)]], zq0_v, sem0)

    acc = jnp.zeros((LANES,), jnp.float32)
    for c in range(NCHUNK):
        zq_v = zq_bufs[c % 2]
        if c + 1 < NCHUNK:
            pltpu.async_copy(
                ebf_hbm.at[idx_v.at[pl.ds((c + 1) * CHUNK, CHUNK)]],
                zq_bufs[(c + 1) % 2], sems[(c + 1) % 2])
        pltpu.sync_copy(x_hbm.at[pl.ds(base + c * CHUNK, CHUNK)], x_v)
        pltpu.make_async_copy(
            ebf_hbm.at[idx_v.at[pl.ds(c * CHUNK, CHUNK)]],
            zq_v, sems[c % 2]).wait()

        def row_body(r, a):
            for j in range(D // LANES):
                sl = pl.ds(j * LANES, LANES)
                zq = zq_v[r, sl]
                b = lax.bitcast_convert_type(zq, jnp.int32)
                b = b + jnp.int32(0x7FFF) + ((b >> 16) & 1)
                b = b & jnp.int32(-65536)
                zq = lax.bitcast_convert_type(b, jnp.float32)
                xv = x_v[r, sl]
                t = zq - xv
                zq_v[r, sl] = xv + t
                a = a + t * t
            return a

        acc = lax.fori_loop(0, CHUNK, row_body, acc)
        pltpu.sync_copy(zq_v, st_hbm.at[pl.ds(base + c * CHUNK, CHUNK)])

    vec_v[...] = acc
    pltpu.sync_copy(vec_v, part_hbm.at[wid])


_sc_quantize = functools.partial(
    pl.kernel,
    mesh=plsc.VectorSubcoreMesh(core_axis_name="c", subcore_axis_name="s"),
    out_type=[
        jax.ShapeDtypeStruct((N, D), jnp.float32),
        jax.ShapeDtypeStruct((NW, LANES), jnp.float32),
    ],
    scratch_types=[
        pltpu.VMEM((BPW,), jnp.int32),
        pltpu.VMEM((CHUNK, D), jnp.float32),
        pltpu.VMEM((CHUNK, D), jnp.float32),
        pltpu.VMEM((CHUNK, D), jnp.float32),
        pltpu.VMEM((LANES,), jnp.float32),
        pltpu.SemaphoreType.DMA,
        pltpu.SemaphoreType.DMA,
    ],
)(_sc_kernel_fn)


def kernel(z_e, embedding):
    B, Dm, H, W = z_e.shape
    z = jnp.transpose(z_e, (0, 2, 3, 1)).reshape(N, D)
    enc, idxo, perp = pl.pallas_call(
        _tc_body,
        grid=(GRID,),
        in_specs=[
            pl.BlockSpec((T, D), lambda i: (i, 0)),
            pl.BlockSpec((K, D), lambda i: (0, 0)),
        ],
        out_specs=[
            pl.BlockSpec((T, K), lambda i: (i, 0)),
            pl.BlockSpec((T, 1), lambda i: (i, 0)),
            pl.BlockSpec((1, 1), lambda i: (0, 0), memory_space=pltpu.SMEM),
        ],
        out_shape=[
            jax.ShapeDtypeStruct((N, K), jnp.float32),
            jax.ShapeDtypeStruct((N, 1), jnp.int32),
            jax.ShapeDtypeStruct((1, 1), jnp.float32),
        ],
        scratch_shapes=[
            pltpu.VMEM((1, K), jnp.float32),
        ],
    )(z, embedding)
    st, loss_parts = _sc_quantize(embedding, idxo.reshape(N), z)
    m = jnp.sum(loss_parts) / jnp.float32(N * D)
    loss = m + COMMITMENT_COST * m
    out = jnp.transpose(st.reshape(B, H, W, Dm), (0, 3, 1, 2))
    return out, loss, perp[0, 0], enc


# speedup vs baseline: 1.1680x; 1.0203x over previous
import functools

import jax
import jax.numpy as jnp
from jax import lax
from jax.experimental import pallas as pl
from jax.experimental.pallas import tpu as pltpu
from jax.experimental.pallas import tpu_sc as plsc

K = 8192
D = 256
N = 8192
T = 256
GRID = N // T
COMMITMENT_COST = 0.25

NC = 2
NS = 16
NW = NC * NS
BPW = N // NW
CHUNK = 64
NCHUNK = BPW // CHUNK
LANES = 16


def _tc_body(x_ref, emb_ref, enc_ref, idx_ref, perp_ref, hist_acc):
    i = pl.program_id(0)
    x = x_ref[...]
    e = emb_ref[...]

    @pl.when(i == 0)
    def _init():
        hist_acc[...] = jnp.zeros_like(hist_acc)

    x_norm = jnp.sum(x * x, axis=1, keepdims=True)
    e_norm = jnp.sum(e * e, axis=1)
    x2b = (x + x).astype(jnp.bfloat16)
    eb = e.astype(jnp.bfloat16)
    prod2 = jax.lax.dot_general(x2b, eb, (((1,), (1,)), ((), ())),
                                preferred_element_type=jnp.float32)
    dist = (x_norm + e_norm[None, :]) - prod2

    mn = jnp.min(dist, axis=1, keepdims=True)
    iota = jax.lax.broadcasted_iota(jnp.int32, (T, K), 1)
    idx = jnp.min(jnp.where(dist == mn, iota, K), axis=1)
    idx_ref[...] = idx[:, None]
    enc = (iota == idx[:, None]).astype(jnp.float32)
    enc_ref[...] = enc
    hist_acc[...] += jnp.sum(enc, axis=0)[None, :]

    @pl.when(i == GRID - 1)
    def _fini():
        avg = hist_acc[...] * jnp.float32(1.0 / N)
        ent = jnp.sum(avg * jnp.log(avg + 1e-10))
        perp_ref[0, 0] = jnp.exp(-ent)


def _sc_kernel_fn(ebf_hbm, idx_hbm, x_hbm, st_hbm, part_hbm,
                  idx_v, zq0_v, zq1_v, x0_v, x1_v, vec_v,
                  sem0, sem1, xsem0, xsem1):
    wid = lax.axis_index("s") * NC + lax.axis_index("c")
    base = wid * BPW
    pltpu.sync_copy(idx_hbm.at[pl.ds(base, BPW)], idx_v)

    zq_bufs = (zq0_v, zq1_v)
    x_bufs = (x0_v, x1_v)
    sems = (sem0, sem1)
    xsems = (xsem0, xsem1)
    pltpu.async_copy(ebf_hbm.at[idx_v.at[pl.ds(0, CHUNK)]], zq0_v, sem0)
    pltpu.async_copy(x_hbm.at[pl.ds(base, CHUNK)], x0_v, xsem0)

    acc = jnp.zeros((LANES,), jnp.float32)
    for c in range(NCHUNK):
        zq_v = zq_bufs[c % 2]
        x_v = x_bufs[c % 2]
        if c + 1 < NCHUNK:
            pltpu.async_copy(
                ebf_hbm.at[idx_v.at[pl.ds((c + 1) * CHUNK, CHUNK)]],
                zq_bufs[(c + 1) % 2], sems[(c + 1) % 2])
            pltpu.async_copy(
                x_hbm.at[pl.ds(base + (c + 1) * CHUNK, CHUNK)],
                x_bufs[(c + 1) % 2], xsems[(c + 1) % 2])
        pltpu.make_async_copy(
            ebf_hbm.at[idx_v.at[pl.ds(c * CHUNK, CHUNK)]],
            zq_v, sems[c % 2]).wait()
        pltpu.make_async_copy(
            x_hbm.at[pl.ds(base + c * CHUNK, CHUNK)],
            x_v, xsems[c % 2]).wait()

        def row_body(r, a):
            for j in range(D // LANES):
                sl = pl.ds(j * LANES, LANES)
                zq = zq_v[r, sl]
                b = lax.bitcast_convert_type(zq, jnp.int32)
                b = b + jnp.int32(0x7FFF) + ((b >> 16) & 1)
                b = b & jnp.int32(-65536)
                zq = lax.bitcast_convert_type(b, jnp.float32)
                xv = x_v[r, sl]
                t = zq - xv
                zq_v[r, sl] = xv + t
                a = a + t * t
            return a

        acc = lax.fori_loop(0, CHUNK, row_body, acc)
        pltpu.sync_copy(zq_v, st_hbm.at[pl.ds(base + c * CHUNK, CHUNK)])

    vec_v[...] = acc
    pltpu.sync_copy(vec_v, part_hbm.at[wid])


_sc_quantize = functools.partial(
    pl.kernel,
    mesh=plsc.VectorSubcoreMesh(core_axis_name="c", subcore_axis_name="s"),
    out_type=[
        jax.ShapeDtypeStruct((N, D), jnp.float32),
        jax.ShapeDtypeStruct((NW, LANES), jnp.float32),
    ],
    scratch_types=[
        pltpu.VMEM((BPW,), jnp.int32),
        pltpu.VMEM((CHUNK, D), jnp.float32),
        pltpu.VMEM((CHUNK, D), jnp.float32),
        pltpu.VMEM((CHUNK, D), jnp.float32),
        pltpu.VMEM((CHUNK, D), jnp.float32),
        pltpu.VMEM((LANES,), jnp.float32),
        pltpu.SemaphoreType.DMA,
        pltpu.SemaphoreType.DMA,
        pltpu.SemaphoreType.DMA,
        pltpu.SemaphoreType.DMA,
    ],
)(_sc_kernel_fn)


def kernel(z_e, embedding):
    B, Dm, H, W = z_e.shape
    z = jnp.transpose(z_e, (0, 2, 3, 1)).reshape(N, D)
    enc, idxo, perp = pl.pallas_call(
        _tc_body,
        grid=(GRID,),
        in_specs=[
            pl.BlockSpec((T, D), lambda i: (i, 0)),
            pl.BlockSpec((K, D), lambda i: (0, 0)),
        ],
        out_specs=[
            pl.BlockSpec((T, K), lambda i: (i, 0)),
            pl.BlockSpec((T, 1), lambda i: (i, 0)),
            pl.BlockSpec((1, 1), lambda i: (0, 0), memory_space=pltpu.SMEM),
        ],
        out_shape=[
            jax.ShapeDtypeStruct((N, K), jnp.float32),
            jax.ShapeDtypeStruct((N, 1), jnp.int32),
            jax.ShapeDtypeStruct((1, 1), jnp.float32),
        ],
        scratch_shapes=[
            pltpu.VMEM((1, K), jnp.float32),
        ],
    )(z, embedding)
    st, loss_parts = _sc_quantize(embedding, idxo.reshape(N), z)
    m = jnp.sum(loss_parts) / jnp.float32(N * D)
    loss = m + COMMITMENT_COST * m
    out = jnp.transpose(st.reshape(B, H, W, Dm), (0, 3, 1, 2))
    return out, loss, perp[0, 0], enc
